# baseline (speedup 1.0000x reference)
import functools

import jax
import jax.numpy as jnp
from jax import lax
from jax.experimental import pallas as pl
from jax.experimental.pallas import tpu as pltpu
from jax.experimental.pallas import tpu_sc as plsc

VOCAB = 100000
SEQ_LEN = 2048
EMBED = 128
BATCH = 4

NC = 2
NS = 16
NW = NC * NS
P_PER_W = SEQ_LEN // NW
ROWS_PER_W = BATCH * P_PER_W
LANES = 16
SCALE = 11.31370849898476


def _sc_embed(idx, token_table, pos_table):
  mesh = plsc.VectorSubcoreMesh(core_axis_name="c", subcore_axis_name="s")

  @functools.partial(
      pl.kernel,
      mesh=mesh,
      out_type=jax.ShapeDtypeStruct((BATCH, SEQ_LEN, EMBED), jnp.float32),
      scratch_types=[
          pltpu.VMEM((BATCH, P_PER_W), jnp.int32),
          pltpu.VMEM((ROWS_PER_W, EMBED), jnp.float32),
          pltpu.VMEM((P_PER_W, EMBED), jnp.float32),
          pltpu.SemaphoreType.DMA,
          pltpu.SemaphoreType.DMA((BATCH,)),
          pltpu.SemaphoreType.DMA((BATCH,)),
      ],
  )
  def k(idx_hbm, tok_hbm, pos_hbm, out_hbm, idx_v, out_v, pos_v, isem,
        gsem, wsem):
    wid = lax.axis_index("s") * NC + lax.axis_index("c")
    p0 = wid * P_PER_W
    idx_copies = [
        pltpu.async_copy(idx_hbm.at[b, pl.ds(p0, P_PER_W)], idx_v.at[b],
                         isem)
        for b in range(BATCH)
    ]
    pos_copy = pltpu.async_copy(pos_hbm.at[pl.ds(p0, P_PER_W)], pos_v, isem)
    for c in idx_copies:
      c.wait()
    gathers = [
        pltpu.async_copy(
            tok_hbm.at[idx_v.at[b]],
            out_v.at[pl.ds(b * P_PER_W, P_PER_W)],
            gsem.at[b],
        ) for b in range(BATCH)
    ]
    pos_copy.wait()

    writes = []
    for b in range(BATCH):
      gathers[b].wait()

      def row(r, carry):
        for j in range(EMBED // LANES):
          csl = pl.ds(j * LANES, LANES)
          o = (b * P_PER_W + r, csl)
          out_v[o] = out_v[o] * SCALE + pos_v[r, csl]
        return carry

      lax.fori_loop(0, P_PER_W, row, 0, unroll=2)
      writes.append(
          pltpu.async_copy(
              out_v.at[pl.ds(b * P_PER_W, P_PER_W)],
              out_hbm.at[b].at[pl.ds(p0, P_PER_W)],
              wsem.at[b],
          ))
    for w in writes:
      w.wait()

  return k(idx, token_table, pos_table)


def kernel(inputs, token_table, pos_table):
  return _sc_embed(inputs.astype(jnp.int32), token_table, pos_table)
